# baseline (speedup 1.0000x reference)
import jax
import jax.numpy as jnp
from jax.experimental import pallas as pl
from jax.experimental.pallas import tpu as pltpu

_pallas_call = pl.pallas_call

NSAMPLE = 8
GN_GROUPS = 16
GN_EPS = 1e-5
Q = 2048


def _k1_body(x1_ref, x1t_ref, x2_ref, f1_ref, f2_ref, w1f_ref, w1i_ref,
             y1_ref, st_ref, d2s_ref, t2s_ref):
    t = pl.program_id(1)
    x1 = x1_ref[0]
    x2 = x2_ref[0]
    n1 = x1.shape[1]

    @pl.when(t == 0)
    def _():
        x1t = x1t_ref[0]
        t2s_ref[...] = (x1t[:, 0:1] * x1t[:, 0:1]
                        + x1t[:, 1:2] * x1t[:, 1:2]) \
            + x1t[:, 2:3] * x1t[:, 2:3]

    d2s_ref[...] = jax.lax.dot_general(x1 * -2.0, x2,
                                       (((0,), (0,)), ((), ())),
                                       preferred_element_type=jnp.float32)
    q2 = (x2[0:1, :] * x2[0:1, :] + x2[1:2, :] * x2[1:2, :]) \
        + x2[2:3, :] * x2[2:3, :]
    sort8 = [(0, 1), (2, 3), (4, 5), (6, 7),
             (0, 2), (1, 3), (4, 6), (5, 7),
             (1, 2), (5, 6),
             (0, 4), (1, 5), (2, 6), (3, 7),
             (2, 4), (3, 5),
             (1, 2), (3, 4), (5, 6)]
    bitonic8 = [(0, 4), (1, 5), (2, 6), (3, 7),
                (0, 2), (1, 3), (4, 6), (5, 7),
                (0, 1), (2, 3), (4, 5), (6, 7)]

    def _grp(g, bs):
        s = [t2s_ref[pl.ds(g * 64 + k * 8, 8), :]
             + d2s_ref[pl.ds(g * 64 + k * 8, 8), :] for k in range(8)]
        for i, j in sort8:
            lo = jnp.minimum(s[i], s[j])
            hi = jnp.maximum(s[i], s[j])
            s[i], s[j] = lo, hi
        m = [jnp.minimum(bs[i], s[7 - i]) for i in range(8)]
        for i, j in bitonic8:
            lo = jnp.minimum(m[i], m[j])
            hi = jnp.maximum(m[i], m[j])
            m[i], m[j] = lo, hi
        return tuple(m)

    init = tuple(jnp.full((8, Q), jnp.inf, jnp.float32)
                 for _ in range(NSAMPLE))
    bs = jax.lax.fori_loop(0, n1 // 64, _grp, init, unroll=8)
    allb = jnp.concatenate(bs, axis=0)
    for _ in range(NSAMPLE - 1):
        m = jnp.min(allb, axis=0, keepdims=True)
        allb = jnp.where(allb == m, jnp.inf, allb)
    th = jnp.min(allb, axis=0, keepdims=True)
    u = t2s_ref[...] + d2s_ref[...]
    d2 = q2 + u
    wf = jax.lax.rsqrt(jnp.maximum(d2, 1e-12))
    wacc = jnp.where(u <= th, wf, 0.0)
    wsum = jnp.sum(wacc, axis=0, keepdims=True)
    interp = jax.lax.dot_general(f1_ref[0], wacc, (((1,), (0,)), ((), ())),
                                 preferred_element_type=jnp.float32) / wsum
    y1 = (jax.lax.dot_general(w1f_ref[...], f2_ref[0], (((1,), (0,)), ((), ())),
                              preferred_element_type=jnp.float32)
          + jax.lax.dot_general(w1i_ref[...], interp, (((1,), (0,)), ((), ())),
                                preferred_element_type=jnp.float32))
    y1_ref[0] = y1
    s = jnp.sum(y1, axis=1, keepdims=True)
    sq = jnp.sum(y1 * y1, axis=1, keepdims=True)
    lio = jax.lax.broadcasted_iota(jnp.int32, (y1.shape[0], 128), 1)
    val = jnp.where(lio == 0, s, 0.0) + jnp.where(lio == 1, sq, 0.0)

    @pl.when(t == 0)
    def _():
        st_ref[0] = val

    @pl.when(t != 0)
    def _():
        st_ref[0] = st_ref[0] + val


def _k2_body(y1_ref, sc_ref, sh_ref, w2_ref, y2_ref, st_ref):
    t = pl.program_id(1)
    a = y1_ref[0] * sc_ref[0] + sh_ref[0]
    a = jnp.where(a >= 0, a, 0.1 * a)
    y2 = jax.lax.dot_general(w2_ref[...], a, (((1,), (0,)), ((), ())),
                             preferred_element_type=jnp.float32)
    y2_ref[0] = y2
    s = jnp.sum(y2, axis=1, keepdims=True)
    sq = jnp.sum(y2 * y2, axis=1, keepdims=True)
    lio = jax.lax.broadcasted_iota(jnp.int32, (y2.shape[0], 128), 1)
    val = jnp.where(lio == 0, s, 0.0) + jnp.where(lio == 1, sq, 0.0)

    @pl.when(t == 0)
    def _():
        st_ref[0] = val

    @pl.when(t != 0)
    def _():
        st_ref[0] = st_ref[0] + val


def _k3_body(y2_ref, sc_ref, sh_ref, out_ref):
    a = y2_ref[0] * sc_ref[0] + sh_ref[0]
    out_ref[0] = jnp.where(a >= 0, a, 0.1 * a)


def _gn_scale_shift(stats, b, g, be, n_pts):
    C = stats.shape[1]
    s = stats[:, :, 0] + n_pts * b[None, :]
    q = stats[:, :, 1] + 2.0 * b[None, :] * stats[:, :, 0] + n_pts * b[None, :] ** 2
    cpg = C // GN_GROUPS
    n = cpg * n_pts
    gs = s.reshape(-1, GN_GROUPS, cpg).sum(-1) / n
    gq = q.reshape(-1, GN_GROUPS, cpg).sum(-1) / n
    var = gq - gs * gs
    rstd = jax.lax.rsqrt(var + GN_EPS)
    mean_c = jnp.repeat(gs, cpg, axis=1)
    rstd_c = jnp.repeat(rstd, cpg, axis=1)
    scale = rstd_c * g[None, :]
    shift = (b[None, :] - mean_c) * rstd_c * g[None, :] + be[None, :]
    return scale[:, :, None], shift[:, :, None]


def kernel(xyz1, xyz2, feat1, feat2, W1, b1, g1, be1, W2, b2, g2, be2):
    B, _, N1 = xyz1.shape
    N2 = xyz2.shape[2]
    C1 = feat1.shape[1]
    C2 = feat2.shape[1]
    CO = W1.shape[0]
    T = N2 // Q

    pad = jnp.zeros((B, 5, N1), jnp.float32)
    x1p = jnp.concatenate([xyz1, pad], axis=1)
    x1tp = jnp.transpose(x1p, (0, 2, 1))
    x2p = jnp.concatenate([xyz2, jnp.zeros((B, 5, N2), jnp.float32)], axis=1)
    W1f = W1[:, :C2]
    W1i = W1[:, C2:]

    y1, st1 = _pallas_call(
        _k1_body,
        grid=(B, T),
        in_specs=[
            pl.BlockSpec((1, 8, N1), lambda b, t: (b, 0, 0)),
            pl.BlockSpec((1, N1, 8), lambda b, t: (b, 0, 0)),
            pl.BlockSpec((1, 8, Q), lambda b, t: (b, 0, t)),
            pl.BlockSpec((1, C1, N1), lambda b, t: (b, 0, 0)),
            pl.BlockSpec((1, C2, Q), lambda b, t: (b, 0, t)),
            pl.BlockSpec((CO, C2), lambda b, t: (0, 0)),
            pl.BlockSpec((CO, C1), lambda b, t: (0, 0)),
        ],
        out_specs=[
            pl.BlockSpec((1, CO, Q), lambda b, t: (b, 0, t)),
            pl.BlockSpec((1, CO, 128), lambda b, t: (b, 0, 0)),
        ],
        out_shape=[
            jax.ShapeDtypeStruct((B, CO, N2), jnp.float32),
            jax.ShapeDtypeStruct((B, CO, 128), jnp.float32),
        ],
        scratch_shapes=[pltpu.VMEM((N1, Q), jnp.float32),
                        pltpu.VMEM((N1, 1), jnp.float32)],
    )(x1p, x1tp, x2p, feat1, feat2, W1f, W1i)

    sc1, sh1 = _gn_scale_shift(st1, b1, g1, be1, N2)

    y2, st2 = _pallas_call(
        _k2_body,
        grid=(B, T),
        in_specs=[
            pl.BlockSpec((1, CO, Q), lambda b, t: (b, 0, t)),
            pl.BlockSpec((1, CO, 1), lambda b, t: (b, 0, 0)),
            pl.BlockSpec((1, CO, 1), lambda b, t: (b, 0, 0)),
            pl.BlockSpec((CO, CO), lambda b, t: (0, 0)),
        ],
        out_specs=[
            pl.BlockSpec((1, CO, Q), lambda b, t: (b, 0, t)),
            pl.BlockSpec((1, CO, 128), lambda b, t: (b, 0, 0)),
        ],
        out_shape=[
            jax.ShapeDtypeStruct((B, CO, N2), jnp.float32),
            jax.ShapeDtypeStruct((B, CO, 128), jnp.float32),
        ],
    )(y1, sc1, sh1, W2)

    sc2, sh2 = _gn_scale_shift(st2, b2, g2, be2, N2)

    out = _pallas_call(
        _k3_body,
        grid=(B, T),
        in_specs=[
            pl.BlockSpec((1, CO, Q), lambda b, t: (b, 0, t)),
            pl.BlockSpec((1, CO, 1), lambda b, t: (b, 0, 0)),
            pl.BlockSpec((1, CO, 1), lambda b, t: (b, 0, 0)),
        ],
        out_specs=pl.BlockSpec((1, CO, Q), lambda b, t: (b, 0, t)),
        out_shape=jax.ShapeDtypeStruct((B, CO, N2), jnp.float32),
    )(y2, sc2, sh2)

    return out


# speedup vs baseline: 1.1048x; 1.1048x over previous
import jax
import jax.numpy as jnp
from jax.experimental import pallas as pl
from jax.experimental.pallas import tpu as pltpu

_pallas_call = pl.pallas_call

NSAMPLE = 8
GN_GROUPS = 16
GN_EPS = 1e-5
Q = 2048


def _k1_body(x1_ref, x1t_ref, x2_ref, f1_ref, f2_ref, w1f_ref, w1i_ref,
             y1_ref, st_ref, d2s_ref, t2s_ref):
    t = pl.program_id(1)
    x1 = x1_ref[0]
    x2 = x2_ref[0]
    n1 = x1.shape[1]

    @pl.when(t == 0)
    def _():
        x1t = x1t_ref[0]
        t2s_ref[...] = (x1t[:, 0:1] * x1t[:, 0:1]
                        + x1t[:, 1:2] * x1t[:, 1:2]) \
            + x1t[:, 2:3] * x1t[:, 2:3]

    e = jax.lax.dot_general(x1 * -2.0, x2, (((0,), (0,)), ((), ())),
                            preferred_element_type=jnp.float32)
    t2 = t2s_ref[...]
    q2 = (x2[0:1, :] * x2[0:1, :] + x2[1:2, :] * x2[1:2, :]) \
        + x2[2:3, :] * x2[2:3, :]
    d2s_ref[...] = (q2 + t2) + e
    sort8 = [(0, 1), (2, 3), (4, 5), (6, 7),
             (0, 2), (1, 3), (4, 6), (5, 7),
             (1, 2), (5, 6),
             (0, 4), (1, 5), (2, 6), (3, 7),
             (2, 4), (3, 5),
             (1, 2), (3, 4), (5, 6)]
    bitonic8 = [(0, 4), (1, 5), (2, 6), (3, 7),
                (0, 2), (1, 3), (4, 6), (5, 7),
                (0, 1), (2, 3), (4, 5), (6, 7)]

    def _grp(g, bs):
        s = [d2s_ref[pl.ds(g * 64 + k * 8, 8), :] for k in range(8)]
        for i, j in sort8:
            lo = jnp.minimum(s[i], s[j])
            hi = jnp.maximum(s[i], s[j])
            s[i], s[j] = lo, hi
        m = [jnp.minimum(bs[i], s[7 - i]) for i in range(8)]
        for i, j in bitonic8:
            lo = jnp.minimum(m[i], m[j])
            hi = jnp.maximum(m[i], m[j])
            m[i], m[j] = lo, hi
        return tuple(m)

    init = tuple(jnp.full((8, Q), jnp.inf, jnp.float32)
                 for _ in range(NSAMPLE))
    bs = jax.lax.fori_loop(0, n1 // 64, _grp, init, unroll=8)
    allb = jnp.concatenate(bs, axis=0)
    for _ in range(NSAMPLE - 1):
        m = jnp.min(allb, axis=0, keepdims=True)
        allb = jnp.where(allb == m, jnp.inf, allb)
    th = jnp.min(allb, axis=0, keepdims=True)
    d2 = d2s_ref[...]
    wf = jax.lax.rsqrt(jnp.maximum(d2, 1e-12))
    wacc = jnp.where(d2 <= th, wf, 0.0)
    wsum = jnp.sum(wacc, axis=0, keepdims=True)
    interp = jax.lax.dot_general(f1_ref[0], wacc, (((1,), (0,)), ((), ())),
                                 preferred_element_type=jnp.float32) / wsum
    y1 = (jax.lax.dot_general(w1f_ref[...], f2_ref[0], (((1,), (0,)), ((), ())),
                              preferred_element_type=jnp.float32)
          + jax.lax.dot_general(w1i_ref[...], interp, (((1,), (0,)), ((), ())),
                                preferred_element_type=jnp.float32))
    y1_ref[0] = y1
    s = jnp.sum(y1, axis=1, keepdims=True)
    sq = jnp.sum(y1 * y1, axis=1, keepdims=True)
    lio = jax.lax.broadcasted_iota(jnp.int32, (y1.shape[0], 128), 1)
    val = jnp.where(lio == 0, s, 0.0) + jnp.where(lio == 1, sq, 0.0)

    @pl.when(t == 0)
    def _():
        st_ref[0] = val

    @pl.when(t != 0)
    def _():
        st_ref[0] = st_ref[0] + val


def _k2_body(y1_ref, sc_ref, sh_ref, w2_ref, y2_ref, st_ref):
    t = pl.program_id(1)
    a = y1_ref[0] * sc_ref[0] + sh_ref[0]
    a = jnp.where(a >= 0, a, 0.1 * a)
    y2 = jax.lax.dot_general(w2_ref[...], a, (((1,), (0,)), ((), ())),
                             preferred_element_type=jnp.float32)
    y2_ref[0] = y2
    s = jnp.sum(y2, axis=1, keepdims=True)
    sq = jnp.sum(y2 * y2, axis=1, keepdims=True)
    lio = jax.lax.broadcasted_iota(jnp.int32, (y2.shape[0], 128), 1)
    val = jnp.where(lio == 0, s, 0.0) + jnp.where(lio == 1, sq, 0.0)

    @pl.when(t == 0)
    def _():
        st_ref[0] = val

    @pl.when(t != 0)
    def _():
        st_ref[0] = st_ref[0] + val


def _k3_body(y2_ref, sc_ref, sh_ref, out_ref):
    a = y2_ref[0] * sc_ref[0] + sh_ref[0]
    out_ref[0] = jnp.where(a >= 0, a, 0.1 * a)


def _gn_scale_shift(stats, b, g, be, n_pts):
    C = stats.shape[1]
    s = stats[:, :, 0] + n_pts * b[None, :]
    q = stats[:, :, 1] + 2.0 * b[None, :] * stats[:, :, 0] + n_pts * b[None, :] ** 2
    cpg = C // GN_GROUPS
    n = cpg * n_pts
    gs = s.reshape(-1, GN_GROUPS, cpg).sum(-1) / n
    gq = q.reshape(-1, GN_GROUPS, cpg).sum(-1) / n
    var = gq - gs * gs
    rstd = jax.lax.rsqrt(var + GN_EPS)
    mean_c = jnp.repeat(gs, cpg, axis=1)
    rstd_c = jnp.repeat(rstd, cpg, axis=1)
    scale = rstd_c * g[None, :]
    shift = (b[None, :] - mean_c) * rstd_c * g[None, :] + be[None, :]
    return scale[:, :, None], shift[:, :, None]


def kernel(xyz1, xyz2, feat1, feat2, W1, b1, g1, be1, W2, b2, g2, be2):
    B, _, N1 = xyz1.shape
    N2 = xyz2.shape[2]
    C1 = feat1.shape[1]
    C2 = feat2.shape[1]
    CO = W1.shape[0]
    T = N2 // Q

    pad = jnp.zeros((B, 5, N1), jnp.float32)
    x1p = jnp.concatenate([xyz1, pad], axis=1)
    x1tp = jnp.transpose(x1p, (0, 2, 1))
    x2p = jnp.concatenate([xyz2, jnp.zeros((B, 5, N2), jnp.float32)], axis=1)
    W1f = W1[:, :C2]
    W1i = W1[:, C2:]

    y1, st1 = _pallas_call(
        _k1_body,
        grid=(B, T),
        in_specs=[
            pl.BlockSpec((1, 8, N1), lambda b, t: (b, 0, 0)),
            pl.BlockSpec((1, N1, 8), lambda b, t: (b, 0, 0)),
            pl.BlockSpec((1, 8, Q), lambda b, t: (b, 0, t)),
            pl.BlockSpec((1, C1, N1), lambda b, t: (b, 0, 0)),
            pl.BlockSpec((1, C2, Q), lambda b, t: (b, 0, t)),
            pl.BlockSpec((CO, C2), lambda b, t: (0, 0)),
            pl.BlockSpec((CO, C1), lambda b, t: (0, 0)),
        ],
        out_specs=[
            pl.BlockSpec((1, CO, Q), lambda b, t: (b, 0, t)),
            pl.BlockSpec((1, CO, 128), lambda b, t: (b, 0, 0)),
        ],
        out_shape=[
            jax.ShapeDtypeStruct((B, CO, N2), jnp.float32),
            jax.ShapeDtypeStruct((B, CO, 128), jnp.float32),
        ],
        scratch_shapes=[pltpu.VMEM((N1, Q), jnp.float32),
                        pltpu.VMEM((N1, 1), jnp.float32)],
    )(x1p, x1tp, x2p, feat1, feat2, W1f, W1i)

    sc1, sh1 = _gn_scale_shift(st1, b1, g1, be1, N2)

    y2, st2 = _pallas_call(
        _k2_body,
        grid=(B, T),
        in_specs=[
            pl.BlockSpec((1, CO, Q), lambda b, t: (b, 0, t)),
            pl.BlockSpec((1, CO, 1), lambda b, t: (b, 0, 0)),
            pl.BlockSpec((1, CO, 1), lambda b, t: (b, 0, 0)),
            pl.BlockSpec((CO, CO), lambda b, t: (0, 0)),
        ],
        out_specs=[
            pl.BlockSpec((1, CO, Q), lambda b, t: (b, 0, t)),
            pl.BlockSpec((1, CO, 128), lambda b, t: (b, 0, 0)),
        ],
        out_shape=[
            jax.ShapeDtypeStruct((B, CO, N2), jnp.float32),
            jax.ShapeDtypeStruct((B, CO, 128), jnp.float32),
        ],
    )(y1, sc1, sh1, W2)

    sc2, sh2 = _gn_scale_shift(st2, b2, g2, be2, N2)

    out = _pallas_call(
        _k3_body,
        grid=(B, T),
        in_specs=[
            pl.BlockSpec((1, CO, Q), lambda b, t: (b, 0, t)),
            pl.BlockSpec((1, CO, 1), lambda b, t: (b, 0, 0)),
            pl.BlockSpec((1, CO, 1), lambda b, t: (b, 0, 0)),
        ],
        out_specs=pl.BlockSpec((1, CO, Q), lambda b, t: (b, 0, t)),
        out_shape=jax.ShapeDtypeStruct((B, CO, N2), jnp.float32),
    )(y2, sc2, sh2)

    return out
